# baseline (speedup 1.0000x reference)
import jax
import jax.numpy as jnp
from jax.experimental import pallas as pl
from jax.experimental.pallas import tpu as pltpu

_EPS = 1e-12
_LOW, _HIGH = 0.02, 0.98
_ITERS = 10

_S = 128
_LANES = 128
_TBL = _S * _LANES


def _fold(row, t):
    return jnp.concatenate(
        [row[:, (t * 8 + s) * _LANES:(t * 8 + s + 1) * _LANES]
         for s in range(8)], axis=0)


def _body(m2_ref, m3_ref, w1_ref, w2_ref, w3_ref, mus_ref, v_ref):
    w1 = w1_ref[...]
    w2 = w2_ref[...]
    w3 = w3_ref[...]
    ones = jnp.ones((1, _TBL), jnp.float32)
    x2 = jnp.concatenate([m2_ref[...], ones], axis=0)

    h = jnp.dot(w1, x2, preferred_element_type=jnp.float32)
    h = jnp.maximum(h, 0.0)
    h = jnp.concatenate([h, ones], axis=0)
    h = jnp.maximum(jnp.dot(w2, h, preferred_element_type=jnp.float32), 0.0)
    h = jnp.concatenate([h, ones], axis=0)
    pars = jnp.dot(w3, h, preferred_element_type=jnp.float32)

    span = _HIGH - _LOW
    sqs = lambda z: _LOW + span * (1.0 / (1.0 + jnp.exp(-z)))

    for t in range(_S // 8):
        sl = slice(t * 8, (t + 1) * 8)
        p = [_fold(pars[e:e + 1, :], t) for e in range(9)]

        a00 = jnp.exp(p[0])
        a01 = jnp.exp(p[1])
        a10 = jnp.exp(p[2])
        a11 = jnp.exp(p[3])
        a02 = jnp.sqrt(a00 * a01)
        a12 = jnp.sqrt(a10 * a11)
        a20 = jnp.sqrt(a00 * a10)
        a21 = jnp.sqrt(a01 * a11)
        a22 = jnp.sqrt(a20 * a21)

        shm0 = sqs(p[4])
        shm1 = sqs(p[5])
        shf0 = sqs(p[6])
        shf1 = sqs(p[7])
        vv = jnp.exp(p[8])

        m0 = m3_ref[0, sl, :]
        m1 = m3_ref[1, sl, :]
        m2 = m3_ref[2, sl, :]
        f0 = m3_ref[3, sl, :]
        f1 = m3_ref[4, sl, :]
        f2 = m3_ref[5, sl, :]
        r = [m0 * shm0, m1 * shm1, m2]
        c = [f0 * shf0, f1 * shf1, f2]

        A = [[a00, a01, a02], [a10, a11, a12], [a20, a21, a22]]

        for _ in range(_ITERS):
            for i in range(3):
                s = A[i][0] + A[i][1] + A[i][2]
                f = r[i] / (s + _EPS)
                A[i] = [A[i][0] * f, A[i][1] * f, A[i][2] * f]
            for j in range(3):
                s = A[0][j] + A[1][j] + A[2][j]
                g = c[j] / (s + _EPS)
                for i in range(3):
                    A[i][j] = A[i][j] * g

        mum0_0 = m0 * (1.0 - shm0)
        mum0_1 = m1 * (1.0 - shm1)
        mu0f_0 = f0 * (1.0 - shf0)
        mu0f_1 = f1 * (1.0 - shf1)
        zero = jnp.zeros((8, _LANES), jnp.float32)

        outs = [A[0][0], A[0][1], A[0][2], mum0_0,
                A[1][0], A[1][1], A[1][2], mum0_1,
                A[2][0], A[2][1], A[2][2], zero,
                mu0f_0, mu0f_1, zero, zero]
        for e in range(16):
            mus_ref[e // 4, e % 4, sl, :] = outs[e]
        v_ref[sl, :] = vv


def kernel(margins, W1, b1, W2, b2, W3, b3):
    Bn = margins.shape[0]
    rows = Bn // _LANES
    nb = rows // _S

    mt = margins.T
    mt3 = mt.reshape(8, rows, _LANES)
    w1a = jnp.concatenate([W1.T, b1[:, None]], axis=1)
    w2a = jnp.concatenate([W2.T, b2[:, None]], axis=1)
    w3a = jnp.concatenate([W3.T, b3[:, None]], axis=1)

    musT, vT = pl.pallas_call(
        _body,
        grid=(nb,),
        in_specs=[
            pl.BlockSpec((8, _TBL), lambda i: (0, i)),
            pl.BlockSpec((8, _S, _LANES), lambda i: (0, i, 0)),
            pl.BlockSpec((32, 9), lambda i: (0, 0)),
            pl.BlockSpec((16, 33), lambda i: (0, 0)),
            pl.BlockSpec((9, 17), lambda i: (0, 0)),
        ],
        out_specs=[
            pl.BlockSpec((4, 4, _S, _LANES), lambda i: (0, 0, i, 0)),
            pl.BlockSpec((_S, _LANES), lambda i: (i, 0)),
        ],
        out_shape=[
            jax.ShapeDtypeStruct((4, 4, rows, _LANES), jnp.float32),
            jax.ShapeDtypeStruct((rows, _LANES), jnp.float32),
        ],
        compiler_params=pltpu.CompilerParams(
            dimension_semantics=("parallel",),
        ),
    )(mt, mt3, w1a, w2a, w3a)

    mus = jnp.transpose(musT.reshape(4, 4, Bn), (2, 0, 1))
    V = vT.reshape(Bn)
    return mus, V


# speedup vs baseline: 1.0356x; 1.0356x over previous
import jax
import jax.numpy as jnp
from jax.experimental import pallas as pl
from jax.experimental.pallas import tpu as pltpu

_EPS = 1e-12
_LOW, _HIGH = 0.02, 0.98
_ITERS = 10

_S = 128
_LANES = 128
_TBL = _S * _LANES


def _fold(row, t):
    return jnp.concatenate(
        [row[:, (t * 8 + s) * _LANES:(t * 8 + s + 1) * _LANES]
         for s in range(8)], axis=0)


def _body(m2_ref, m3_ref, w1_ref, w2_ref, w3_ref, mus_ref, v_ref):
    w1 = w1_ref[...]
    w2 = w2_ref[...]
    w3 = w3_ref[...]
    span = _HIGH - _LOW
    sqs = lambda z: _LOW + span * (1.0 / (1.0 + jnp.exp(-z)))

    _CHUNK = _TBL
    _CS = _CHUNK // _LANES

    ones = jnp.ones((1, _CHUNK), jnp.float32)
    pars_chunks = []
    for q in range(_TBL // _CHUNK):
        cs = slice(q * _CHUNK, (q + 1) * _CHUNK)
        x2 = jnp.concatenate([m2_ref[:, cs], ones], axis=0)
        h = jnp.dot(w1, x2, preferred_element_type=jnp.float32)
        h = jnp.maximum(h, 0.0)
        h = jnp.concatenate([h, ones], axis=0)
        h = jnp.maximum(jnp.dot(w2, h, preferred_element_type=jnp.float32), 0.0)
        h = jnp.concatenate([h, ones], axis=0)
        pars_chunks.append(
            jnp.dot(w3, h, preferred_element_type=jnp.float32))

    for t in range(_S // 8):
        sl = slice(t * 8, (t + 1) * 8)
        pq = pars_chunks[t // (_CS // 8)]
        p = [_fold(pq[e:e + 1, :], t % (_CS // 8)) for e in range(9)]

        a00 = jnp.exp(p[0])
        a01 = jnp.exp(p[1])
        a10 = jnp.exp(p[2])
        a11 = jnp.exp(p[3])
        a02 = jnp.sqrt(a00 * a01)
        a12 = jnp.sqrt(a10 * a11)
        a20 = jnp.sqrt(a00 * a10)
        a21 = jnp.sqrt(a01 * a11)
        a22 = jnp.sqrt(a20 * a21)

        shm0 = sqs(p[4])
        shm1 = sqs(p[5])
        shf0 = sqs(p[6])
        shf1 = sqs(p[7])
        vv = jnp.exp(p[8])

        m0 = m3_ref[0, sl, :]
        m1 = m3_ref[1, sl, :]
        m2 = m3_ref[2, sl, :]
        f0 = m3_ref[3, sl, :]
        f1 = m3_ref[4, sl, :]
        f2 = m3_ref[5, sl, :]
        r = [m0 * shm0, m1 * shm1, m2]
        c = [f0 * shf0, f1 * shf1, f2]

        A = [[a00, a01, a02], [a10, a11, a12], [a20, a21, a22]]

        for _ in range(_ITERS):
            for i in range(3):
                s = A[i][0] + A[i][1] + A[i][2]
                f = r[i] / s
                A[i] = [A[i][0] * f, A[i][1] * f, A[i][2] * f]
            for j in range(3):
                s = A[0][j] + A[1][j] + A[2][j]
                g = c[j] / s
                for i in range(3):
                    A[i][j] = A[i][j] * g

        mum0_0 = m0 - r[0]
        mum0_1 = m1 - r[1]
        mu0f_0 = f0 - c[0]
        mu0f_1 = f1 - c[1]
        zero = jnp.zeros((8, _LANES), jnp.float32)

        outs = [A[0][0], A[0][1], A[0][2], mum0_0,
                A[1][0], A[1][1], A[1][2], mum0_1,
                A[2][0], A[2][1], A[2][2], zero,
                mu0f_0, mu0f_1, zero, zero]
        for e in range(16):
            mus_ref[e // 4, e % 4, sl, :] = outs[e]
        v_ref[sl, :] = vv


def kernel(margins, W1, b1, W2, b2, W3, b3):
    Bn = margins.shape[0]
    rows = Bn // _LANES
    nb = rows // _S

    mt = margins.T
    mt3 = mt.reshape(8, rows, _LANES)
    w1a = jnp.concatenate([W1.T, b1[:, None]], axis=1)
    w2a = jnp.concatenate([W2.T, b2[:, None]], axis=1)
    w3a = jnp.concatenate([W3.T, b3[:, None]], axis=1)

    musT, vT = pl.pallas_call(
        _body,
        grid=(nb,),
        in_specs=[
            pl.BlockSpec((8, _TBL), lambda i: (0, i)),
            pl.BlockSpec((8, _S, _LANES), lambda i: (0, i, 0)),
            pl.BlockSpec((32, 9), lambda i: (0, 0)),
            pl.BlockSpec((16, 33), lambda i: (0, 0)),
            pl.BlockSpec((9, 17), lambda i: (0, 0)),
        ],
        out_specs=[
            pl.BlockSpec((4, 4, _S, _LANES), lambda i: (0, 0, i, 0)),
            pl.BlockSpec((_S, _LANES), lambda i: (i, 0)),
        ],
        out_shape=[
            jax.ShapeDtypeStruct((4, 4, rows, _LANES), jnp.float32),
            jax.ShapeDtypeStruct((rows, _LANES), jnp.float32),
        ],
        compiler_params=pltpu.CompilerParams(
            dimension_semantics=("parallel",),
        ),
    )(mt, mt3, w1a, w2a, w3a)

    mus = jnp.transpose(musT.reshape(4, 4, Bn), (2, 0, 1))
    V = vT.reshape(Bn)
    return mus, V


# speedup vs baseline: 1.0709x; 1.0341x over previous
import jax
import jax.numpy as jnp
from jax.experimental import pallas as pl
from jax.experimental.pallas import tpu as pltpu

_EPS = 1e-12
_LOW, _HIGH = 0.02, 0.98
_ITERS = 10

_S = 128
_LANES = 128
_TBL = _S * _LANES


def _fold(row, t):
    return jnp.concatenate(
        [row[:, (t * 8 + s) * _LANES:(t * 8 + s + 1) * _LANES]
         for s in range(8)], axis=0)


def _fold8(tile, t):
    v = [tile[:, (t * 8 + c) * _LANES:(t * 8 + c + 1) * _LANES]
         for c in range(8)]
    iota = jax.lax.broadcasted_iota(jnp.int32, (8, _LANES), 0)
    for k in (1, 2, 4):
        m = (iota & k) == 0
        nv = list(v)
        for i in range(8):
            if i & k:
                continue
            j = i | k
            a, b = v[i], v[j]
            nv[i] = jnp.where(m, a, pltpu.roll(b, k, 0))
            nv[j] = jnp.where(m, pltpu.roll(a, 8 - k, 0), b)
        v = nv
    return v


def _body(m2_ref, m3_ref, w1_ref, w2_ref, w3_ref, mus_ref, v_ref):
    w1 = w1_ref[...]
    w2 = w2_ref[...]
    w3 = w3_ref[...]
    span = _HIGH - _LOW
    sqs = lambda z: _LOW + span * (1.0 / (1.0 + jnp.exp(-z)))

    _CHUNK = _TBL
    _CS = _CHUNK // _LANES

    ones = jnp.ones((1, _CHUNK), jnp.float32)
    pars_chunks = []
    for q in range(_TBL // _CHUNK):
        cs = slice(q * _CHUNK, (q + 1) * _CHUNK)
        x2 = jnp.concatenate([m2_ref[:, cs], ones], axis=0)
        h = jnp.dot(w1, x2, preferred_element_type=jnp.float32)
        h = jnp.maximum(h, 0.0)
        h = jnp.concatenate([h, ones], axis=0)
        h = jnp.maximum(jnp.dot(w2, h, preferred_element_type=jnp.float32), 0.0)
        h = jnp.concatenate([h, ones], axis=0)
        pars_chunks.append(
            jnp.dot(w3, h, preferred_element_type=jnp.float32))

    for t in range(_S // 8):
        sl = slice(t * 8, (t + 1) * 8)
        pq = pars_chunks[t // (_CS // 8)]
        tq = t % (_CS // 8)
        p = _fold8(pq[0:8, :], tq) + [_fold(pq[8:9, :], tq)]

        a00 = jnp.exp(p[0])
        a01 = jnp.exp(p[1])
        a10 = jnp.exp(p[2])
        a11 = jnp.exp(p[3])
        a02 = jnp.sqrt(a00 * a01)
        a12 = jnp.sqrt(a10 * a11)
        a20 = jnp.sqrt(a00 * a10)
        a21 = jnp.sqrt(a01 * a11)
        a22 = jnp.sqrt(a20 * a21)

        shm0 = sqs(p[4])
        shm1 = sqs(p[5])
        shf0 = sqs(p[6])
        shf1 = sqs(p[7])
        vv = jnp.exp(p[8])

        m0 = m3_ref[0, sl, :]
        m1 = m3_ref[1, sl, :]
        m2 = m3_ref[2, sl, :]
        f0 = m3_ref[3, sl, :]
        f1 = m3_ref[4, sl, :]
        f2 = m3_ref[5, sl, :]
        r = [m0 * shm0, m1 * shm1, m2]
        c = [f0 * shf0, f1 * shf1, f2]

        A = [[a00, a01, a02], [a10, a11, a12], [a20, a21, a22]]

        for _ in range(_ITERS):
            for i in range(3):
                s = A[i][0] + A[i][1] + A[i][2]
                f = r[i] / s
                A[i] = [A[i][0] * f, A[i][1] * f, A[i][2] * f]
            for j in range(3):
                s = A[0][j] + A[1][j] + A[2][j]
                g = c[j] / s
                for i in range(3):
                    A[i][j] = A[i][j] * g

        mum0_0 = m0 - r[0]
        mum0_1 = m1 - r[1]
        mu0f_0 = f0 - c[0]
        mu0f_1 = f1 - c[1]
        zero = jnp.zeros((8, _LANES), jnp.float32)

        outs = [A[0][0], A[0][1], A[0][2], mum0_0,
                A[1][0], A[1][1], A[1][2], mum0_1,
                A[2][0], A[2][1], A[2][2], zero,
                mu0f_0, mu0f_1, zero, zero]
        for e in range(16):
            mus_ref[e // 4, e % 4, sl, :] = outs[e]
        v_ref[sl, :] = vv


def kernel(margins, W1, b1, W2, b2, W3, b3):
    Bn = margins.shape[0]
    rows = Bn // _LANES
    nb = rows // _S

    mt = margins.T
    mt3 = mt.reshape(8, rows, _LANES)
    w1a = jnp.concatenate([W1.T, b1[:, None]], axis=1)
    w2a = jnp.concatenate([W2.T, b2[:, None]], axis=1)
    w3a = jnp.concatenate([W3.T, b3[:, None]], axis=1)

    musT, vT = pl.pallas_call(
        _body,
        grid=(nb,),
        in_specs=[
            pl.BlockSpec((8, _TBL), lambda i: (0, i)),
            pl.BlockSpec((8, _S, _LANES), lambda i: (0, i, 0)),
            pl.BlockSpec((32, 9), lambda i: (0, 0)),
            pl.BlockSpec((16, 33), lambda i: (0, 0)),
            pl.BlockSpec((9, 17), lambda i: (0, 0)),
        ],
        out_specs=[
            pl.BlockSpec((4, 4, _S, _LANES), lambda i: (0, 0, i, 0)),
            pl.BlockSpec((_S, _LANES), lambda i: (i, 0)),
        ],
        out_shape=[
            jax.ShapeDtypeStruct((4, 4, rows, _LANES), jnp.float32),
            jax.ShapeDtypeStruct((rows, _LANES), jnp.float32),
        ],
        compiler_params=pltpu.CompilerParams(
            dimension_semantics=("parallel",),
        ),
    )(mt, mt3, w1a, w2a, w3a)

    mus = jnp.transpose(musT.reshape(4, 4, Bn), (2, 0, 1))
    V = vT.reshape(Bn)
    return mus, V


# speedup vs baseline: 1.4573x; 1.3607x over previous
import jax
import jax.numpy as jnp
from jax.experimental import pallas as pl
from jax.experimental.pallas import tpu as pltpu

_EPS = 1e-12
_LOW, _HIGH = 0.02, 0.98
_ITERS = 10

_S = 128
_LANES = 128
_TBL = _S * _LANES


def _fold(row, t):
    return jnp.concatenate(
        [row[:, (t * 8 + s) * _LANES:(t * 8 + s + 1) * _LANES]
         for s in range(8)], axis=0)


def _bfly8(v):
    iota = jax.lax.broadcasted_iota(jnp.int32, (8, _LANES), 0)
    for k in (1, 2, 4):
        m = (iota & k) == 0
        nv = list(v)
        for i in range(8):
            if i & k:
                continue
            j = i | k
            a, b = v[i], v[j]
            nv[i] = jnp.where(m, a, pltpu.roll(b, k, 0))
            nv[j] = jnp.where(m, pltpu.roll(a, 8 - k, 0), b)
        v = nv
    return v


def _fold8(tile, t):
    return _bfly8([tile[:, (t * 8 + c) * _LANES:(t * 8 + c + 1) * _LANES]
                   for c in range(8)])


def _body(m2_ref, m3_ref, w1_ref, w2_ref, w3_ref, mus_ref, v_ref):
    w1 = w1_ref[...]
    w2 = w2_ref[...]
    w3 = w3_ref[...]
    span = _HIGH - _LOW
    sqs = lambda z: _LOW + span * (1.0 / (1.0 + jnp.exp(-z)))

    _CHUNK = _TBL
    _CS = _CHUNK // _LANES

    ones = jnp.ones((1, _CHUNK), jnp.float32)
    pars_chunks = []
    for q in range(_TBL // _CHUNK):
        cs = slice(q * _CHUNK, (q + 1) * _CHUNK)
        x2 = jnp.concatenate([m2_ref[:, cs], ones], axis=0)
        h = jnp.dot(w1, x2, preferred_element_type=jnp.float32)
        h = jnp.maximum(h, 0.0)
        h = jnp.concatenate([h, ones], axis=0)
        h = jnp.maximum(jnp.dot(w2, h, preferred_element_type=jnp.float32), 0.0)
        h = jnp.concatenate([h, ones], axis=0)
        pars_chunks.append(
            jnp.dot(w3, h, preferred_element_type=jnp.float32))

    for t in range(_S // 8):
        sl = slice(t * 8, (t + 1) * 8)
        pq = pars_chunks[t // (_CS // 8)]
        tq = t % (_CS // 8)
        p = _fold8(pq[0:8, :], tq) + [_fold(pq[8:9, :], tq)]

        a00 = jnp.exp(p[0])
        a01 = jnp.exp(p[1])
        a10 = jnp.exp(p[2])
        a11 = jnp.exp(p[3])
        a02 = jnp.sqrt(a00 * a01)
        a12 = jnp.sqrt(a10 * a11)
        a20 = jnp.sqrt(a00 * a10)
        a21 = jnp.sqrt(a01 * a11)
        a22 = jnp.sqrt(a20 * a21)

        shm0 = sqs(p[4])
        shm1 = sqs(p[5])
        shf0 = sqs(p[6])
        shf1 = sqs(p[7])
        vv = jnp.exp(p[8])

        m0 = m3_ref[0, sl, :]
        m1 = m3_ref[1, sl, :]
        m2 = m3_ref[2, sl, :]
        f0 = m3_ref[3, sl, :]
        f1 = m3_ref[4, sl, :]
        f2 = m3_ref[5, sl, :]
        r = [m0 * shm0, m1 * shm1, m2]
        c = [f0 * shf0, f1 * shf1, f2]

        A = [[a00, a01, a02], [a10, a11, a12], [a20, a21, a22]]

        for _ in range(_ITERS):
            for i in range(3):
                s = A[i][0] + A[i][1] + A[i][2]
                f = r[i] / s
                A[i] = [A[i][0] * f, A[i][1] * f, A[i][2] * f]
            for j in range(3):
                s = A[0][j] + A[1][j] + A[2][j]
                g = c[j] / s
                for i in range(3):
                    A[i][j] = A[i][j] * g

        mum0_0 = m0 - r[0]
        mum0_1 = m1 - r[1]
        mu0f_0 = f0 - c[0]
        mu0f_1 = f1 - c[1]
        zero = jnp.zeros((8, _LANES), jnp.float32)

        outs = [A[0][0], A[0][1], A[0][2], mum0_0,
                A[1][0], A[1][1], A[1][2], mum0_1,
                A[2][0], A[2][1], A[2][2], zero,
                mu0f_0, mu0f_1, zero, zero]
        va = _bfly8(outs[0:8])
        vb = _bfly8(outs[8:16])
        for cidx in range(8):
            cs = slice((t * 8 + cidx) * _LANES, (t * 8 + cidx + 1) * _LANES)
            mus_ref[0, :, cs] = va[cidx][0:4, :]
            mus_ref[1, :, cs] = va[cidx][4:8, :]
            mus_ref[2, :, cs] = vb[cidx][0:4, :]
            mus_ref[3, :, cs] = vb[cidx][4:8, :]
        v_ref[sl, :] = vv


def kernel(margins, W1, b1, W2, b2, W3, b3):
    Bn = margins.shape[0]
    rows = Bn // _LANES
    nb = rows // _S

    mt = margins.T
    mt3 = mt.reshape(8, rows, _LANES)
    w1a = jnp.concatenate([W1.T, b1[:, None]], axis=1)
    w2a = jnp.concatenate([W2.T, b2[:, None]], axis=1)
    w3a = jnp.concatenate([W3.T, b3[:, None]], axis=1)

    musT, vT = pl.pallas_call(
        _body,
        grid=(nb,),
        in_specs=[
            pl.BlockSpec((8, _TBL), lambda i: (0, i)),
            pl.BlockSpec((8, _S, _LANES), lambda i: (0, i, 0)),
            pl.BlockSpec((32, 9), lambda i: (0, 0)),
            pl.BlockSpec((16, 33), lambda i: (0, 0)),
            pl.BlockSpec((9, 17), lambda i: (0, 0)),
        ],
        out_specs=[
            pl.BlockSpec((4, 4, _TBL), lambda i: (0, 0, i)),
            pl.BlockSpec((_S, _LANES), lambda i: (i, 0)),
        ],
        out_shape=[
            jax.ShapeDtypeStruct((4, 4, Bn), jnp.float32),
            jax.ShapeDtypeStruct((rows, _LANES), jnp.float32),
        ],
        compiler_params=pltpu.CompilerParams(
            dimension_semantics=("parallel",),
        ),
    )(mt, mt3, w1a, w2a, w3a)

    mus = jnp.transpose(musT, (2, 0, 1))
    V = vT.reshape(Bn)
    return mus, V


# speedup vs baseline: 1.7585x; 1.2067x over previous
import jax
import jax.numpy as jnp
from jax.experimental import pallas as pl
from jax.experimental.pallas import tpu as pltpu

_EPS = 1e-12
_LOW, _HIGH = 0.02, 0.98
_ITERS = 10

_S = 128
_LANES = 128
_TBL = _S * _LANES


def _fold(row, t):
    return jnp.concatenate(
        [row[:, (t * 8 + s) * _LANES:(t * 8 + s + 1) * _LANES]
         for s in range(8)], axis=0)


def _bfly8(v):
    iota = jax.lax.broadcasted_iota(jnp.int32, (8, _LANES), 0)
    for k in (1, 2, 4):
        m = (iota & k) == 0
        nv = list(v)
        for i in range(8):
            if i & k:
                continue
            j = i | k
            a, b = v[i], v[j]
            nv[i] = jnp.where(m, a, pltpu.roll(b, k, 0))
            nv[j] = jnp.where(m, pltpu.roll(a, 8 - k, 0), b)
        v = nv
    return v


def _fold8(tile, t):
    return _bfly8([tile[:, (t * 8 + c) * _LANES:(t * 8 + c + 1) * _LANES]
                   for c in range(8)])


def _body(m2_ref, w1_ref, w2_ref, w3_ref, mus_ref, v_ref):
    w1 = w1_ref[...]
    w2 = w2_ref[...]
    w3 = w3_ref[...]
    span = _HIGH - _LOW
    sqs = lambda z: _LOW + span * (1.0 / (1.0 + jnp.exp(-z)))

    _CHUNK = _TBL
    _CS = _CHUNK // _LANES

    ones = jnp.ones((1, _CHUNK), jnp.float32)
    pars_chunks = []
    for q in range(_TBL // _CHUNK):
        cs = slice(q * _CHUNK, (q + 1) * _CHUNK)
        x2 = jnp.concatenate([m2_ref[:, cs], ones], axis=0)
        h = jnp.dot(w1, x2, preferred_element_type=jnp.float32)
        h = jnp.maximum(h, 0.0)
        h = jnp.concatenate([h, ones], axis=0)
        h = jnp.maximum(jnp.dot(w2, h, preferred_element_type=jnp.float32), 0.0)
        h = jnp.concatenate([h, ones], axis=0)
        pars_chunks.append(
            jnp.dot(w3, h, preferred_element_type=jnp.float32))

    for t in range(_S // 8):
        sl = slice(t * 8, (t + 1) * 8)
        pq = pars_chunks[t // (_CS // 8)]
        tq = t % (_CS // 8)
        p = _fold8(pq[0:8, :], tq) + [_fold(pq[8:9, :], tq)]

        a00 = jnp.exp(p[0])
        a01 = jnp.exp(p[1])
        a10 = jnp.exp(p[2])
        a11 = jnp.exp(p[3])
        a02 = jnp.sqrt(a00 * a01)
        a12 = jnp.sqrt(a10 * a11)
        a20 = jnp.sqrt(a00 * a10)
        a21 = jnp.sqrt(a01 * a11)
        a22 = jnp.sqrt(a20 * a21)

        shm0 = sqs(p[4])
        shm1 = sqs(p[5])
        shf0 = sqs(p[6])
        shf1 = sqs(p[7])
        vv = jnp.exp(p[8])

        mm = _fold8(m2_ref[...], t)
        m0, m1, m2, f0, f1, f2 = mm[0], mm[1], mm[2], mm[3], mm[4], mm[5]
        r = [m0 * shm0, m1 * shm1, m2]
        c = [f0 * shf0, f1 * shf1, f2]

        A = [[a00, a01, a02], [a10, a11, a12], [a20, a21, a22]]

        for _ in range(_ITERS):
            for i in range(3):
                s = A[i][0] + A[i][1] + A[i][2]
                f = r[i] / s
                A[i] = [A[i][0] * f, A[i][1] * f, A[i][2] * f]
            for j in range(3):
                s = A[0][j] + A[1][j] + A[2][j]
                g = c[j] / s
                for i in range(3):
                    A[i][j] = A[i][j] * g

        mum0_0 = m0 - r[0]
        mum0_1 = m1 - r[1]
        mu0f_0 = f0 - c[0]
        mu0f_1 = f1 - c[1]
        zero = jnp.zeros((8, _LANES), jnp.float32)

        outs = [A[0][0], A[0][1], A[0][2], mum0_0,
                A[1][0], A[1][1], A[1][2], mum0_1,
                A[2][0], A[2][1], A[2][2], zero,
                mu0f_0, mu0f_1, zero, zero]
        va = _bfly8(outs[0:8])
        vb = _bfly8(outs[8:16])
        for cidx in range(8):
            cs = slice((t * 8 + cidx) * _LANES, (t * 8 + cidx + 1) * _LANES)
            mus_ref[0, :, cs] = va[cidx][0:4, :]
            mus_ref[1, :, cs] = va[cidx][4:8, :]
            mus_ref[2, :, cs] = vb[cidx][0:4, :]
            mus_ref[3, :, cs] = vb[cidx][4:8, :]
        v_ref[sl, :] = vv


def kernel(margins, W1, b1, W2, b2, W3, b3):
    Bn = margins.shape[0]
    rows = Bn // _LANES
    nb = rows // _S

    mt = margins.T
    w1a = jnp.concatenate([W1.T, b1[:, None]], axis=1)
    w2a = jnp.concatenate([W2.T, b2[:, None]], axis=1)
    w3a = jnp.concatenate([W3.T, b3[:, None]], axis=1)

    musT, vT = pl.pallas_call(
        _body,
        grid=(nb,),
        in_specs=[
            pl.BlockSpec((8, _TBL), lambda i: (0, i)),
            pl.BlockSpec((32, 9), lambda i: (0, 0)),
            pl.BlockSpec((16, 33), lambda i: (0, 0)),
            pl.BlockSpec((9, 17), lambda i: (0, 0)),
        ],
        out_specs=[
            pl.BlockSpec((4, 4, _TBL), lambda i: (0, 0, i)),
            pl.BlockSpec((_S, _LANES), lambda i: (i, 0)),
        ],
        out_shape=[
            jax.ShapeDtypeStruct((4, 4, Bn), jnp.float32),
            jax.ShapeDtypeStruct((rows, _LANES), jnp.float32),
        ],
        compiler_params=pltpu.CompilerParams(
            dimension_semantics=("parallel",),
        ),
    )(mt, w1a, w2a, w3a)

    mus = jnp.transpose(musT, (2, 0, 1))
    V = vT.reshape(Bn)
    return mus, V


# speedup vs baseline: 1.7740x; 1.0088x over previous
import jax
import jax.numpy as jnp
from jax.experimental import pallas as pl
from jax.experimental.pallas import tpu as pltpu

_EPS = 1e-12
_LOW, _HIGH = 0.02, 0.98
_ITERS = 10

_S = 256
_LANES = 128
_TBL = _S * _LANES


def _fold(row, t):
    return jnp.concatenate(
        [row[:, (t * 8 + s) * _LANES:(t * 8 + s + 1) * _LANES]
         for s in range(8)], axis=0)


def _bfly8(v):
    iota = jax.lax.broadcasted_iota(jnp.int32, (8, _LANES), 0)
    for k in (1, 2, 4):
        m = (iota & k) == 0
        nv = list(v)
        for i in range(8):
            if i & k:
                continue
            j = i | k
            a, b = v[i], v[j]
            nv[i] = jnp.where(m, a, pltpu.roll(b, k, 0))
            nv[j] = jnp.where(m, pltpu.roll(a, 8 - k, 0), b)
        v = nv
    return v


def _fold8(tile, t):
    return _bfly8([tile[:, (t * 8 + c) * _LANES:(t * 8 + c + 1) * _LANES]
                   for c in range(8)])


def _body(m2_ref, w1_ref, w2_ref, w3_ref, mus_ref, v_ref):
    w1 = w1_ref[...]
    w2 = w2_ref[...]
    w3 = w3_ref[...]
    span = _HIGH - _LOW
    sqs = lambda z: _LOW + span * (1.0 / (1.0 + jnp.exp(-z)))

    _CHUNK = _TBL
    _CS = _CHUNK // _LANES

    ones = jnp.ones((1, _CHUNK), jnp.float32)
    pars_chunks = []
    for q in range(_TBL // _CHUNK):
        cs = slice(q * _CHUNK, (q + 1) * _CHUNK)
        x2 = jnp.concatenate([m2_ref[:, cs], ones], axis=0)
        h = jnp.dot(w1, x2, preferred_element_type=jnp.float32)
        h = jnp.maximum(h, 0.0)
        h = jnp.concatenate([h, ones], axis=0)
        h = jnp.maximum(jnp.dot(w2, h, preferred_element_type=jnp.float32), 0.0)
        h = jnp.concatenate([h, ones], axis=0)
        pars_chunks.append(
            jnp.dot(w3, h, preferred_element_type=jnp.float32))

    for t in range(_S // 8):
        sl = slice(t * 8, (t + 1) * 8)
        pq = pars_chunks[t // (_CS // 8)]
        tq = t % (_CS // 8)
        p = _fold8(pq[0:8, :], tq) + [_fold(pq[8:9, :], tq)]

        a00 = jnp.exp(p[0])
        a01 = jnp.exp(p[1])
        a10 = jnp.exp(p[2])
        a11 = jnp.exp(p[3])
        a02 = jnp.sqrt(a00 * a01)
        a12 = jnp.sqrt(a10 * a11)
        a20 = jnp.sqrt(a00 * a10)
        a21 = jnp.sqrt(a01 * a11)
        a22 = jnp.sqrt(a20 * a21)

        shm0 = sqs(p[4])
        shm1 = sqs(p[5])
        shf0 = sqs(p[6])
        shf1 = sqs(p[7])
        vv = jnp.exp(p[8])

        mm = _fold8(m2_ref[...], t)
        m0, m1, m2, f0, f1, f2 = mm[0], mm[1], mm[2], mm[3], mm[4], mm[5]
        r = [m0 * shm0, m1 * shm1, m2]
        c = [f0 * shf0, f1 * shf1, f2]

        A = [[a00, a01, a02], [a10, a11, a12], [a20, a21, a22]]

        for _ in range(_ITERS):
            for i in range(3):
                s = A[i][0] + A[i][1] + A[i][2]
                f = r[i] / s
                A[i] = [A[i][0] * f, A[i][1] * f, A[i][2] * f]
            for j in range(3):
                s = A[0][j] + A[1][j] + A[2][j]
                g = c[j] / s
                for i in range(3):
                    A[i][j] = A[i][j] * g

        mum0_0 = m0 - r[0]
        mum0_1 = m1 - r[1]
        mu0f_0 = f0 - c[0]
        mu0f_1 = f1 - c[1]
        zero = jnp.zeros((8, _LANES), jnp.float32)

        outs = [A[0][0], A[0][1], A[0][2], mum0_0,
                A[1][0], A[1][1], A[1][2], mum0_1,
                A[2][0], A[2][1], A[2][2], zero,
                mu0f_0, mu0f_1, zero, zero]
        va = _bfly8(outs[0:8])
        vb = _bfly8(outs[8:16])
        for cidx in range(8):
            cs = slice((t * 8 + cidx) * _LANES, (t * 8 + cidx + 1) * _LANES)
            mus_ref[0, :, cs] = va[cidx][0:4, :]
            mus_ref[1, :, cs] = va[cidx][4:8, :]
            mus_ref[2, :, cs] = vb[cidx][0:4, :]
            mus_ref[3, :, cs] = vb[cidx][4:8, :]
        v_ref[sl, :] = vv


def kernel(margins, W1, b1, W2, b2, W3, b3):
    Bn = margins.shape[0]
    rows = Bn // _LANES
    nb = rows // _S

    mt = margins.T
    w1a = jnp.concatenate([W1.T, b1[:, None]], axis=1)
    w2a = jnp.concatenate([W2.T, b2[:, None]], axis=1)
    w3a = jnp.concatenate([W3.T, b3[:, None]], axis=1)

    musT, vT = pl.pallas_call(
        _body,
        grid=(nb,),
        in_specs=[
            pl.BlockSpec((8, _TBL), lambda i: (0, i)),
            pl.BlockSpec((32, 9), lambda i: (0, 0)),
            pl.BlockSpec((16, 33), lambda i: (0, 0)),
            pl.BlockSpec((9, 17), lambda i: (0, 0)),
        ],
        out_specs=[
            pl.BlockSpec((4, 4, _TBL), lambda i: (0, 0, i)),
            pl.BlockSpec((_S, _LANES), lambda i: (i, 0)),
        ],
        out_shape=[
            jax.ShapeDtypeStruct((4, 4, Bn), jnp.float32),
            jax.ShapeDtypeStruct((rows, _LANES), jnp.float32),
        ],
        compiler_params=pltpu.CompilerParams(
            dimension_semantics=("parallel",),
        ),
    )(mt, w1a, w2a, w3a)

    mus = jnp.transpose(musT, (2, 0, 1))
    V = vT.reshape(Bn)
    return mus, V


# speedup vs baseline: 1.7975x; 1.0133x over previous
import jax
import jax.numpy as jnp
from jax.experimental import pallas as pl
from jax.experimental.pallas import tpu as pltpu

_EPS = 1e-12
_LOW, _HIGH = 0.02, 0.98
_ITERS = 10

_S = 256
_LANES = 128
_TBL = _S * _LANES


def _fold(row, t):
    return jnp.concatenate(
        [row[:, (t * 8 + s) * _LANES:(t * 8 + s + 1) * _LANES]
         for s in range(8)], axis=0)


def _bfly8(v):
    iota = jax.lax.broadcasted_iota(jnp.int32, (8, _LANES), 0)
    for k in (1, 2, 4):
        m = (iota & k) == 0
        nv = list(v)
        for i in range(8):
            if i & k:
                continue
            j = i | k
            a, b = v[i], v[j]
            nv[i] = jnp.where(m, a, pltpu.roll(b, k, 0))
            nv[j] = jnp.where(m, pltpu.roll(a, 8 - k, 0), b)
        v = nv
    return v


def _fold8(tile, t):
    return _bfly8([tile[:, (t * 8 + c) * _LANES:(t * 8 + c + 1) * _LANES]
                   for c in range(8)])


def _body(m2_ref, w1_ref, w2_ref, w3_ref, mus_ref, v_ref):
    w1 = w1_ref[...]
    w2 = w2_ref[...]
    w3 = w3_ref[...]
    span = _HIGH - _LOW
    sqs = lambda z: _LOW + span * (1.0 / (1.0 + jnp.exp(-z)))

    _CHUNK = _TBL // 4
    _CS = _CHUNK // _LANES

    ones = jnp.ones((1, _CHUNK), jnp.float32)
    pars_chunks = []
    for q in range(_TBL // _CHUNK):
        cs = slice(q * _CHUNK, (q + 1) * _CHUNK)
        x2 = jnp.concatenate([m2_ref[:, cs], ones], axis=0)
        h = jnp.dot(w1, x2, preferred_element_type=jnp.float32)
        h = jnp.maximum(h, 0.0)
        h = jnp.concatenate([h, ones], axis=0)
        h = jnp.dot(w2, h, preferred_element_type=jnp.float32)
        h = jnp.maximum(h, 0.0)
        h = jnp.concatenate([h, ones], axis=0)
        pars_chunks.append(
            jnp.dot(w3, h, preferred_element_type=jnp.float32))

    for t in range(_S // 8):
        sl = slice(t * 8, (t + 1) * 8)
        pq = pars_chunks[t // (_CS // 8)]
        tq = t % (_CS // 8)
        p = _fold8(pq[0:8, :], tq) + [_fold(pq[8:9, :], tq)]

        a00 = jnp.exp(p[0])
        a01 = jnp.exp(p[1])
        a10 = jnp.exp(p[2])
        a11 = jnp.exp(p[3])
        a02 = jnp.sqrt(a00 * a01)
        a12 = jnp.sqrt(a10 * a11)
        a20 = jnp.sqrt(a00 * a10)
        a21 = jnp.sqrt(a01 * a11)
        a22 = jnp.sqrt(a20 * a21)

        shm0 = sqs(p[4])
        shm1 = sqs(p[5])
        shf0 = sqs(p[6])
        shf1 = sqs(p[7])
        vv = jnp.exp(p[8])

        mm = _fold8(m2_ref[...], t)
        m0, m1, m2, f0, f1, f2 = mm[0], mm[1], mm[2], mm[3], mm[4], mm[5]
        r = [m0 * shm0, m1 * shm1, m2]
        c = [f0 * shf0, f1 * shf1, f2]

        A = [[a00, a01, a02], [a10, a11, a12], [a20, a21, a22]]

        for _ in range(_ITERS):
            for i in range(3):
                s = A[i][0] + A[i][1] + A[i][2]
                f = r[i] / s
                A[i] = [A[i][0] * f, A[i][1] * f, A[i][2] * f]
            for j in range(3):
                s = A[0][j] + A[1][j] + A[2][j]
                g = c[j] / s
                for i in range(3):
                    A[i][j] = A[i][j] * g

        mum0_0 = m0 - r[0]
        mum0_1 = m1 - r[1]
        mu0f_0 = f0 - c[0]
        mu0f_1 = f1 - c[1]
        zero = jnp.zeros((8, _LANES), jnp.float32)

        outs = [A[0][0], A[0][1], A[0][2], mum0_0,
                A[1][0], A[1][1], A[1][2], mum0_1,
                A[2][0], A[2][1], A[2][2], zero,
                mu0f_0, mu0f_1, zero, zero]
        va = _bfly8(outs[0:8])
        vb = _bfly8(outs[8:16])
        for cidx in range(8):
            cs = slice((t * 8 + cidx) * _LANES, (t * 8 + cidx + 1) * _LANES)
            mus_ref[0, :, cs] = va[cidx][0:4, :]
            mus_ref[1, :, cs] = va[cidx][4:8, :]
            mus_ref[2, :, cs] = vb[cidx][0:4, :]
            mus_ref[3, :, cs] = vb[cidx][4:8, :]
        v_ref[sl, :] = vv


def kernel(margins, W1, b1, W2, b2, W3, b3):
    Bn = margins.shape[0]
    rows = Bn // _LANES
    nb = rows // _S

    mt = margins.T
    w1a = jnp.concatenate([W1.T, b1[:, None]], axis=1)
    w2a = jnp.concatenate([W2.T, b2[:, None]], axis=1)
    w3a = jnp.concatenate([W3.T, b3[:, None]], axis=1)

    musT, vT = pl.pallas_call(
        _body,
        grid=(nb,),
        in_specs=[
            pl.BlockSpec((8, _TBL), lambda i: (0, i)),
            pl.BlockSpec((32, 9), lambda i: (0, 0)),
            pl.BlockSpec((16, 33), lambda i: (0, 0)),
            pl.BlockSpec((9, 17), lambda i: (0, 0)),
        ],
        out_specs=[
            pl.BlockSpec((4, 4, _TBL), lambda i: (0, 0, i)),
            pl.BlockSpec((_S, _LANES), lambda i: (i, 0)),
        ],
        out_shape=[
            jax.ShapeDtypeStruct((4, 4, Bn), jnp.float32),
            jax.ShapeDtypeStruct((rows, _LANES), jnp.float32),
        ],
        compiler_params=pltpu.CompilerParams(
            dimension_semantics=("parallel",),
        ),
    )(mt, w1a, w2a, w3a)

    mus = jnp.transpose(musT, (2, 0, 1))
    V = vT.reshape(Bn)
    return mus, V
